# baseline (speedup 1.0000x reference)
import functools

import jax
import jax.numpy as jnp
from jax import lax
from jax.experimental import pallas as pl
from jax.experimental.pallas import tpu as pltpu
from jax.experimental.pallas import tpu_sc as plsc


_NC = 2
_NS = 16
_NW = _NC * _NS
_ES = 256


def _sc_transpose(table):
    l_dim, e_dim = table.shape
    lc = l_dim // _NW
    mesh = plsc.VectorSubcoreMesh(core_axis_name="c", subcore_axis_name="s")

    @functools.partial(
        pl.kernel,
        mesh=mesh,
        out_type=jax.ShapeDtypeStruct((e_dim, l_dim), table.dtype),
        scratch_types=[
            pltpu.VMEM((lc, _ES), jnp.float32),
            pltpu.VMEM((_ES, lc), jnp.float32),
        ],
        compiler_params=pltpu.CompilerParams(
            needs_layout_passes=False,
            use_tc_tiling_on_sc=False,
        ),
    )
    def k(table_hbm, tt_hbm, in_v, out_v):
        wid = lax.axis_index("s") * _NC + lax.axis_index("c")
        l0 = wid * lc
        iota = lax.iota(jnp.int32, 16)
        for eb in range(e_dim // _ES):
            e0 = eb * _ES
            pltpu.sync_copy(table_hbm.at[pl.ds(l0, lc), pl.ds(e0, _ES)], in_v)

            def body(e, carry):
                e_vec = jnp.broadcast_to(e, (16,)).astype(jnp.int32)
                for j in range(lc // 16):
                    l_idx = iota + (j * 16)
                    v = plsc.load_gather(in_v, [l_idx, e_vec])
                    out_v[e, pl.ds(j * 16, 16)] = v
                return carry

            lax.fori_loop(0, _ES, body, 0)
            pltpu.sync_copy(out_v, tt_hbm.at[pl.ds(e0, _ES), pl.ds(l0, lc)])

    return k(table)


_EB = 128


def _tc_add_body(x_ref, tt_ref, o_ref):
    o_ref[...] = x_ref[...] + tt_ref[...][None, :, :]


def _tc_add(x, tt):
    b, e, l = x.shape
    return pl.pallas_call(
        _tc_add_body,
        grid=(e // _EB,),
        in_specs=[
            pl.BlockSpec((b, _EB, l), lambda ei: (0, ei, 0)),
            pl.BlockSpec((_EB, l), lambda ei: (ei, 0)),
        ],
        out_specs=pl.BlockSpec((b, _EB, l), lambda ei: (0, ei, 0)),
        out_shape=jax.ShapeDtypeStruct(x.shape, x.dtype),
    )(x, tt)


def kernel(x, table):
    return _tc_add(x, _sc_transpose(table))


# speedup vs baseline: 1.2708x; 1.2708x over previous
import functools

import jax
import jax.numpy as jnp
from jax import lax
from jax.experimental import pallas as pl
from jax.experimental.pallas import tpu as pltpu
from jax.experimental.pallas import tpu_sc as plsc


_NC = 2
_NS = 16
_NW = _NC * _NS
_ES = 256


def _sc_transpose(table):
    l_dim, e_dim = table.shape
    lc = l_dim // _NW
    mesh = plsc.VectorSubcoreMesh(core_axis_name="c", subcore_axis_name="s")

    @functools.partial(
        pl.kernel,
        mesh=mesh,
        out_type=jax.ShapeDtypeStruct((e_dim, l_dim), table.dtype),
        scratch_types=[
            pltpu.VMEM((lc, _ES), jnp.float32),
            pltpu.VMEM((_ES, lc), jnp.float32),
        ],
        compiler_params=pltpu.CompilerParams(
            needs_layout_passes=False,
            use_tc_tiling_on_sc=False,
        ),
    )
    def k(table_hbm, tt_hbm, in_v, out_v):
        wid = lax.axis_index("s") * _NC + lax.axis_index("c")
        l0 = wid * lc
        iota = lax.iota(jnp.int32, 16)
        for eb in range(e_dim // _ES):
            e0 = eb * _ES
            pltpu.sync_copy(table_hbm.at[pl.ds(l0, lc), pl.ds(e0, _ES)], in_v)

            @plsc.parallel_loop(0, _ES, unroll=4)
            def _(e):
                e_vec = jnp.broadcast_to(e, (16,)).astype(jnp.int32)
                for j in range(lc // 16):
                    l_idx = iota + (j * 16)
                    v = plsc.load_gather(in_v, [l_idx, e_vec])
                    out_v[e, pl.ds(j * 16, 16)] = v
            pltpu.sync_copy(out_v, tt_hbm.at[pl.ds(e0, _ES), pl.ds(l0, lc)])

    return k(table)


_EB = 128


def _tc_add_body(x_ref, tt_ref, o_ref):
    o_ref[...] = x_ref[...] + tt_ref[...][None, :, :]


def _tc_add(x, tt):
    b, e, l = x.shape
    return pl.pallas_call(
        _tc_add_body,
        grid=(e // _EB,),
        in_specs=[
            pl.BlockSpec((b, _EB, l), lambda ei: (0, ei, 0)),
            pl.BlockSpec((_EB, l), lambda ei: (ei, 0)),
        ],
        out_specs=pl.BlockSpec((b, _EB, l), lambda ei: (0, ei, 0)),
        out_shape=jax.ShapeDtypeStruct(x.shape, x.dtype),
    )(x, tt)


def kernel(x, table):
    return _tc_add(x, _sc_transpose(table))


# speedup vs baseline: 1.6057x; 1.2636x over previous
import functools

import jax
import jax.numpy as jnp
from jax import lax
from jax.experimental import pallas as pl
from jax.experimental.pallas import tpu as pltpu
from jax.experimental.pallas import tpu_sc as plsc


_NC = 2
_NS = 16
_NW = _NC * _NS
_ES = 256


def _sc_transpose(table):
    l_dim, e_dim = table.shape
    lc = l_dim // _NW
    mesh = plsc.VectorSubcoreMesh(core_axis_name="c", subcore_axis_name="s")

    @functools.partial(
        pl.kernel,
        mesh=mesh,
        out_type=jax.ShapeDtypeStruct((e_dim, l_dim), table.dtype),
        scratch_types=[
            pltpu.VMEM((lc, _ES), jnp.float32),
            pltpu.VMEM((_ES, lc), jnp.float32),
        ],
        compiler_params=pltpu.CompilerParams(
            needs_layout_passes=False,
        ),
    )
    def k(table_hbm, tt_hbm, in_v, out_v):
        wid = lax.axis_index("s") * _NC + lax.axis_index("c")
        l0 = wid * lc
        iota = lax.iota(jnp.int32, 16)
        for eb in range(e_dim // _ES):
            e0 = eb * _ES
            pltpu.sync_copy(table_hbm.at[pl.ds(l0, lc), pl.ds(e0, _ES)], in_v)

            @plsc.parallel_loop(0, _ES, unroll=4)
            def _(e):
                e_vec = jnp.broadcast_to(e, (16,)).astype(jnp.int32)
                for j in range(lc // 16):
                    l_idx = iota + (j * 16)
                    v = plsc.load_gather(in_v, [l_idx, e_vec])
                    out_v[e, pl.ds(j * 16, 16)] = v
            pltpu.sync_copy(out_v, tt_hbm.at[pl.ds(e0, _ES), pl.ds(l0, lc)])

    return k(table)


_EB = 128


def _tc_add_body(x_ref, tt_ref, o_ref):
    o_ref[...] = x_ref[...] + tt_ref[...][None, :, :]


def _tc_add(x, tt):
    b, e, l = x.shape
    return pl.pallas_call(
        _tc_add_body,
        grid=(e // _EB,),
        in_specs=[
            pl.BlockSpec((b, _EB, l), lambda ei: (0, ei, 0)),
            pl.BlockSpec((_EB, l), lambda ei: (ei, 0)),
        ],
        out_specs=pl.BlockSpec((b, _EB, l), lambda ei: (0, ei, 0)),
        out_shape=jax.ShapeDtypeStruct(x.shape, x.dtype),
    )(x, tt)


def kernel(x, table):
    return _tc_add(x, _sc_transpose(table))


# speedup vs baseline: 3.5352x; 2.2016x over previous
import jax
import jax.numpy as jnp
from jax.experimental import pallas as pl
from jax.experimental.pallas import tpu as pltpu


_EB = 128


def _body(x_ref, t_ref, o_ref, tt_ref):
    b = pl.program_id(1)

    @pl.when(b == 0)
    def _():
        tt_ref[...] = t_ref[...].T

    o_ref[...] = x_ref[...] + tt_ref[...][None, :, :]


def kernel(x, table):
    b, e, l = x.shape
    return pl.pallas_call(
        _body,
        grid=(e // _EB, b),
        in_specs=[
            pl.BlockSpec((1, _EB, l), lambda ei, bi: (bi, ei, 0)),
            pl.BlockSpec((l, _EB), lambda ei, bi: (0, ei)),
        ],
        out_specs=pl.BlockSpec((1, _EB, l), lambda ei, bi: (bi, ei, 0)),
        out_shape=jax.ShapeDtypeStruct(x.shape, x.dtype),
        scratch_shapes=[pltpu.VMEM((_EB, l), jnp.float32)],
    )(x, table)


# speedup vs baseline: 4.5031x; 1.2738x over previous
import jax
import jax.numpy as jnp
from jax.experimental import pallas as pl


_EB = 128
_LB = 4096


def _body(x_ref, t_ref, o_ref):
    t = t_ref[...]
    o_ref[...] = x_ref[...] + t.T[None, :, :]


def kernel(x, table):
    b, e, l = x.shape
    grid = (e // _EB, l // _LB)
    return pl.pallas_call(
        _body,
        grid=grid,
        in_specs=[
            pl.BlockSpec((b, _EB, _LB), lambda ei, li: (0, ei, li)),
            pl.BlockSpec((_LB, _EB), lambda ei, li: (li, ei)),
        ],
        out_specs=pl.BlockSpec((b, _EB, _LB), lambda ei, li: (0, ei, li)),
        out_shape=jax.ShapeDtypeStruct(x.shape, x.dtype),
    )(x, table)
